# baseline (speedup 1.0000x reference)
import jax
import jax.numpy as jnp
from jax.experimental import pallas as pl
from jax.experimental.pallas import tpu as pltpu

_LAMBDA_BOX = 5.0
_CHUNK = 2048


def _dl_kernel(raw_ref, tgt_ref, out_ref, g_scr):
    P = raw_ref.shape[1]
    C = raw_ref.shape[2]
    T = tgt_ref.shape[1]

    sub_i = jax.lax.broadcasted_iota(jnp.int32, (8, C), 0)
    lane_i = jax.lax.broadcasted_iota(jnp.int32, (8, C), 1)
    sel = jnp.where(sub_i == lane_i, 1.0, 0.0)

    tgt = tgt_ref[0]
    tx1 = tgt[:, 0:1]
    ty1 = tgt[:, 1:2]
    tx2 = tgt[:, 2:3]
    ty2 = tgt[:, 3:4]
    area_t = (tx2 - tx1) * (ty2 - ty1)

    def body(i, carry):
        run_max, run_idx, sp_acc = carry
        off = pl.multiple_of(i * _CHUNK, _CHUNK)
        blk = raw_ref[0, pl.ds(off, _CHUNK), :]
        pbct = jax.lax.dot_general(
            sel, blk, (((1,), (1,)), ((), ())),
            precision=jax.lax.Precision.HIGHEST,
            preferred_element_type=jnp.float32)
        px1 = pbct[0:1]
        py1 = pbct[1:2]
        px2 = pbct[2:3]
        py2 = pbct[3:4]
        cf = pbct[4:5]
        w = jnp.maximum(jnp.minimum(px2, tx2) - jnp.maximum(px1, tx1), 0.0)
        h = jnp.maximum(jnp.minimum(py2, ty2) - jnp.maximum(py1, ty1), 0.0)
        inter = w * h
        area_p = (px2 - px1) * (py2 - py1)
        r = inter / (area_p + area_t)
        lmax = jnp.max(r, axis=1, keepdims=True)
        larg = jnp.argmax(r, axis=1, keepdims=True).astype(jnp.int32)
        upd = lmax > run_max
        run_max = jnp.where(upd, lmax, run_max)
        run_idx = jnp.where(upd, larg + off, run_idx)
        sp_acc = sp_acc + jnp.maximum(cf, 0.0) + jnp.log1p(
            jnp.exp(-jnp.abs(cf)))
        return run_max, run_idx, sp_acc

    run_max = jnp.full((T, 1), -jnp.inf, jnp.float32)
    run_idx = jnp.zeros((T, 1), jnp.int32)
    sp_acc = jnp.zeros((1, _CHUNK), jnp.float32)
    run_max, run_idx, sp_acc = jax.lax.fori_loop(
        0, P // _CHUNK, body, (run_max, run_idx, sp_acc))

    ri_row = jnp.swapaxes(run_idx, 0, 1)
    sub_iota = jax.lax.broadcasted_iota(jnp.int32, (8, C), 0)
    for t in range(T):
        idx = ri_row[0, t]
        base = pl.multiple_of((idx >> 3) << 3, 8)
        chunk = raw_ref[0, pl.ds(base, 8), :]
        sel_r = sub_iota == (idx & 7)
        g_scr[t:t + 1, :] = jnp.sum(jnp.where(sel_r, chunk, 0.0), axis=0,
                                    keepdims=True)

    g = g_scr[:, :]

    d = jnp.abs(g[:, 0:4] - tgt[:, 0:4])
    box_loss = jnp.sum(jnp.where(d < 1.0, 0.5 * d * d, d - 0.5),
                       axis=(0, 1), keepdims=True)

    logits = g[:, 5:]
    m = jnp.max(logits, axis=1, keepdims=True)
    lse = m + jnp.log(jnp.sum(jnp.exp(logits - m), axis=1, keepdims=True))
    tcls = tgt[:, 4:5].astype(jnp.int32)
    cls_iota = jax.lax.broadcasted_iota(jnp.int32, (T, C - 5), 1)
    logit_t = jnp.sum(jnp.where(cls_iota == tcls, logits, 0.0), axis=1,
                      keepdims=True)
    cls_loss = jnp.sum(lse - logit_t, axis=(0, 1), keepdims=True)

    x = g[:, 4:5]
    eq = run_idx == ri_row
    li = jax.lax.broadcasted_iota(jnp.int32, (T, T), 1)
    ti = jax.lax.broadcasted_iota(jnp.int32, (T, T), 0)
    dup = jnp.sum(jnp.where(eq & (li < ti), 1.0, 0.0), axis=1,
                  keepdims=True) > 0.0
    conf_sub = jnp.sum(jnp.where(dup, 0.0, x), axis=(0, 1), keepdims=True)
    sp_total = jnp.sum(sp_acc, axis=(0, 1), keepdims=True)

    out_ref[0] = (_LAMBDA_BOX * box_loss + cls_loss
                  + sp_total - conf_sub)


def kernel(raw_preds, targets, epoch):
    del epoch
    B, P, C = raw_preds.shape
    T = targets.shape[1]
    per_sample = pl.pallas_call(
        _dl_kernel,
        grid=(B,),
        in_specs=[
            pl.BlockSpec((1, P, C), lambda b: (b, 0, 0)),
            pl.BlockSpec((1, T, 5), lambda b: (b, 0, 0)),
        ],
        out_specs=pl.BlockSpec((1, 1, 1), lambda b: (b, 0, 0)),
        out_shape=jax.ShapeDtypeStruct((B, 1, 1), jnp.float32),
        scratch_shapes=[
            pltpu.VMEM((T, C), jnp.float32),
        ],
        compiler_params=pltpu.CompilerParams(
            dimension_semantics=("parallel",),
        ),
    )(raw_preds, targets)
    return jnp.sum(per_sample) / B


# speedup vs baseline: 2.8914x; 2.8914x over previous
import jax
import jax.numpy as jnp
from jax.experimental import pallas as pl
from jax.experimental.pallas import tpu as pltpu

_LAMBDA_BOX = 5.0
_CHUNK = 2048
_BB = 8


def _dl_kernel(tcp_ref, tgt_ref, out_ref, rmax_s, ridx_s, sp_s, cap_s):
    C = tcp_ref.shape[0]
    T = tgt_ref.shape[1]
    TB = _BB * T
    nb = pl.program_id(1)
    num_nb = pl.num_programs(1)

    @pl.when(nb == 0)
    def _init():
        rmax_s[:, :, :] = jnp.full((_BB, T, 1), -jnp.inf, jnp.float32)
        ridx_s[:, :, :] = jnp.zeros((_BB, T, 1), jnp.int32)
        sp_s[:, :, :] = jnp.zeros((_BB, 1, _CHUNK), jnp.float32)

    lane_tc = jax.lax.broadcasted_iota(jnp.int32, (T, _CHUNK), 1)

    onehots = []
    upds = []
    for bb in range(_BB):
        tgt = tgt_ref[bb]
        tx1 = tgt[:, 0:1]
        ty1 = tgt[:, 1:2]
        tx2 = tgt[:, 2:3]
        ty2 = tgt[:, 3:4]
        area_t = (tx2 - tx1) * (ty2 - ty1)

        px1 = tcp_ref[0:1, bb, :]
        py1 = tcp_ref[1:2, bb, :]
        px2 = tcp_ref[2:3, bb, :]
        py2 = tcp_ref[3:4, bb, :]
        cf = tcp_ref[4:5, bb, :]
        w = jnp.maximum(jnp.minimum(px2, tx2) - jnp.maximum(px1, tx1), 0.0)
        h = jnp.maximum(jnp.minimum(py2, ty2) - jnp.maximum(py1, ty1), 0.0)
        inter = w * h
        area_p = (px2 - px1) * (py2 - py1)
        r = inter / (area_p + area_t)
        lmax = jnp.max(r, axis=1, keepdims=True)
        larg = jnp.argmax(r, axis=1, keepdims=True).astype(jnp.int32)
        upd = lmax > rmax_s[bb]
        rmax_s[bb] = jnp.where(upd, lmax, rmax_s[bb])
        ridx_s[bb] = jnp.where(upd, larg + nb * _CHUNK, ridx_s[bb])
        sp_s[bb] = (sp_s[bb] + jnp.maximum(cf, 0.0)
                    + jnp.log1p(jnp.exp(-jnp.abs(cf))))
        onehots.append(jnp.where(lane_tc == larg, 1.0, 0.0))
        upds.append(jnp.swapaxes(jnp.where(upd, 1.0, 0.0), 0, 1))

    blk2d = jnp.reshape(tcp_ref[:, :, :], (C * _BB, _CHUNK))
    blk_bf = blk2d.astype(jnp.bfloat16)
    oh_bf = jnp.concatenate(onehots, axis=0).astype(jnp.bfloat16)
    capmm = jax.lax.dot_general(
        blk_bf, oh_bf, (((1,), (1,)), ((), ())),
        preferred_element_type=jnp.float32)
    cap3 = jnp.reshape(capmm, (C, _BB, TB))
    bsel = (jax.lax.broadcasted_iota(jnp.int32, (_BB, TB), 1) // T
            == jax.lax.broadcasted_iota(jnp.int32, (_BB, TB), 0))
    cap_new = jnp.sum(jnp.where(bsel[None], cap3, 0.0), axis=1)
    upd_all = jnp.concatenate(upds, axis=1) > 0.0
    cap_s[:, :] = jnp.where(upd_all, cap_new, cap_s[:, :])

    @pl.when(nb == num_nb - 1)
    def _finish():
        for bb in range(_BB):
            tgt_t = jnp.swapaxes(tgt_ref[bb], 0, 1)
            g = cap_s[:, bb * T:(bb + 1) * T]

            d = jnp.abs(g[0:4, :] - tgt_t[0:4, :])
            box_loss = jnp.sum(jnp.where(d < 1.0, 0.5 * d * d, d - 0.5),
                               axis=(0, 1), keepdims=True)

            logits = g[5:, :]
            m = jnp.max(logits, axis=0, keepdims=True)
            lse = m + jnp.log(jnp.sum(jnp.exp(logits - m), axis=0,
                                      keepdims=True))
            tcls = tgt_t[4:5, :].astype(jnp.int32)
            cls_iota = jax.lax.broadcasted_iota(jnp.int32, (C - 5, T), 0)
            logit_t = jnp.sum(jnp.where(cls_iota == tcls, logits, 0.0),
                              axis=0, keepdims=True)
            cls_loss = jnp.sum(lse - logit_t, axis=(0, 1), keepdims=True)

            x = g[4:5, :]
            ri_col = ridx_s[bb]
            ri_row = jnp.swapaxes(ri_col, 0, 1)
            eq = ri_col == ri_row
            si = jax.lax.broadcasted_iota(jnp.int32, (T, T), 0)
            ti = jax.lax.broadcasted_iota(jnp.int32, (T, T), 1)
            dup = jnp.sum(jnp.where(eq & (si < ti), 1.0, 0.0), axis=0,
                          keepdims=True) > 0.0
            conf_sub = jnp.sum(jnp.where(dup, 0.0, x), axis=(0, 1),
                               keepdims=True)
            sp_total = jnp.sum(sp_s[bb], axis=(0, 1), keepdims=True)

            out_ref[bb] = (_LAMBDA_BOX * box_loss + cls_loss
                           + sp_total - conf_sub)


def kernel(raw_preds, targets, epoch):
    del epoch
    B, P, C = raw_preds.shape
    T = targets.shape[1]
    nb = P // _CHUNK
    tcp = jnp.transpose(raw_preds, (2, 0, 1))
    per_sample = pl.pallas_call(
        _dl_kernel,
        grid=(B // _BB, nb),
        in_specs=[
            pl.BlockSpec((C, _BB, _CHUNK), lambda g, n: (0, g, n)),
            pl.BlockSpec((_BB, T, 5), lambda g, n: (g, 0, 0)),
        ],
        out_specs=pl.BlockSpec((_BB, 1, 1), lambda g, n: (g, 0, 0)),
        out_shape=jax.ShapeDtypeStruct((B, 1, 1), jnp.float32),
        scratch_shapes=[
            pltpu.VMEM((_BB, T, 1), jnp.float32),
            pltpu.VMEM((_BB, T, 1), jnp.int32),
            pltpu.VMEM((_BB, 1, _CHUNK), jnp.float32),
            pltpu.VMEM((C, _BB * T), jnp.float32),
        ],
        compiler_params=pltpu.CompilerParams(
            dimension_semantics=("parallel", "arbitrary"),
        ),
    )(tcp, targets)
    return jnp.sum(per_sample) / B


# speedup vs baseline: 3.1818x; 1.1004x over previous
import jax
import jax.numpy as jnp
from jax.experimental import pallas as pl
from jax.experimental.pallas import tpu as pltpu

_LAMBDA_BOX = 5.0
_CHUNK = 4096
_BB = 8


def _dl_kernel(tcp_ref, tgt_ref, out_ref, rmax_s, ridx_s, sp_s, cap_s):
    C = tcp_ref.shape[0]
    T = tgt_ref.shape[1]
    TB = _BB * T
    nb = pl.program_id(1)
    num_nb = pl.num_programs(1)

    @pl.when(nb == 0)
    def _init():
        rmax_s[:, :, :] = jnp.full((_BB, T, 1), -jnp.inf, jnp.float32)
        ridx_s[:, :, :] = jnp.zeros((_BB, T, 1), jnp.int32)
        sp_s[:, :, :] = jnp.zeros((_BB, 1, _CHUNK), jnp.float32)

    lane_tc = jax.lax.broadcasted_iota(jnp.int32, (T, _CHUNK), 1)

    onehots = []
    upds = []
    for bb in range(_BB):
        tgt = tgt_ref[bb]
        tx1 = tgt[:, 0:1]
        ty1 = tgt[:, 1:2]
        tx2 = tgt[:, 2:3]
        ty2 = tgt[:, 3:4]
        area_t = (tx2 - tx1) * (ty2 - ty1)

        px1 = tcp_ref[0:1, bb, :]
        py1 = tcp_ref[1:2, bb, :]
        px2 = tcp_ref[2:3, bb, :]
        py2 = tcp_ref[3:4, bb, :]
        cf = tcp_ref[4:5, bb, :]
        w = jnp.maximum(jnp.minimum(px2, tx2) - jnp.maximum(px1, tx1), 0.0)
        h = jnp.maximum(jnp.minimum(py2, ty2) - jnp.maximum(py1, ty1), 0.0)
        inter = w * h
        area_p = (px2 - px1) * (py2 - py1)
        r = inter / (area_p + area_t)
        lmax = jnp.max(r, axis=1, keepdims=True)
        larg = jnp.argmax(r, axis=1, keepdims=True).astype(jnp.int32)
        upd = lmax > rmax_s[bb]
        rmax_s[bb] = jnp.where(upd, lmax, rmax_s[bb])
        ridx_s[bb] = jnp.where(upd, larg + nb * _CHUNK, ridx_s[bb])
        sp_s[bb] = (sp_s[bb] + jnp.maximum(cf, 0.0)
                    + jnp.log1p(jnp.exp(-jnp.abs(cf))))
        onehots.append(jnp.where(lane_tc == larg, 1.0, 0.0))
        upds.append(jnp.swapaxes(jnp.where(upd, 1.0, 0.0), 0, 1))

    blk2d = jnp.reshape(tcp_ref[:, :, :], (C * _BB, _CHUNK))
    oh = jnp.concatenate(onehots, axis=0)
    capmm = jax.lax.dot_general(
        blk2d, oh, (((1,), (1,)), ((), ())),
        preferred_element_type=jnp.float32)
    cap3 = jnp.reshape(capmm, (C, _BB, TB))
    bsel = (jax.lax.broadcasted_iota(jnp.int32, (_BB, TB), 1) // T
            == jax.lax.broadcasted_iota(jnp.int32, (_BB, TB), 0))
    cap_new = jnp.sum(jnp.where(bsel[None], cap3, 0.0), axis=1)
    upd_all = jnp.concatenate(upds, axis=1) > 0.0
    cap_s[:, :] = jnp.where(upd_all, cap_new, cap_s[:, :])

    @pl.when(nb == num_nb - 1)
    def _finish():
        for bb in range(_BB):
            tgt_t = jnp.swapaxes(tgt_ref[bb], 0, 1)
            g = cap_s[:, bb * T:(bb + 1) * T]

            d = jnp.abs(g[0:4, :] - tgt_t[0:4, :])
            box_loss = jnp.sum(jnp.where(d < 1.0, 0.5 * d * d, d - 0.5),
                               axis=(0, 1), keepdims=True)

            logits = g[5:, :]
            m = jnp.max(logits, axis=0, keepdims=True)
            lse = m + jnp.log(jnp.sum(jnp.exp(logits - m), axis=0,
                                      keepdims=True))
            tcls = tgt_t[4:5, :].astype(jnp.int32)
            cls_iota = jax.lax.broadcasted_iota(jnp.int32, (C - 5, T), 0)
            logit_t = jnp.sum(jnp.where(cls_iota == tcls, logits, 0.0),
                              axis=0, keepdims=True)
            cls_loss = jnp.sum(lse - logit_t, axis=(0, 1), keepdims=True)

            x = g[4:5, :]
            ri_col = ridx_s[bb]
            ri_row = jnp.swapaxes(ri_col, 0, 1)
            eq = ri_col == ri_row
            si = jax.lax.broadcasted_iota(jnp.int32, (T, T), 0)
            ti = jax.lax.broadcasted_iota(jnp.int32, (T, T), 1)
            dup = jnp.sum(jnp.where(eq & (si < ti), 1.0, 0.0), axis=0,
                          keepdims=True) > 0.0
            conf_sub = jnp.sum(jnp.where(dup, 0.0, x), axis=(0, 1),
                               keepdims=True)
            sp_total = jnp.sum(sp_s[bb], axis=(0, 1), keepdims=True)

            out_ref[bb] = (_LAMBDA_BOX * box_loss + cls_loss
                           + sp_total - conf_sub)


def kernel(raw_preds, targets, epoch):
    del epoch
    B, P, C = raw_preds.shape
    T = targets.shape[1]
    nb = P // _CHUNK
    tcp = jnp.transpose(raw_preds, (2, 0, 1))
    per_sample = pl.pallas_call(
        _dl_kernel,
        grid=(B // _BB, nb),
        in_specs=[
            pl.BlockSpec((C, _BB, _CHUNK), lambda g, n: (0, g, n)),
            pl.BlockSpec((_BB, T, 5), lambda g, n: (g, 0, 0)),
        ],
        out_specs=pl.BlockSpec((_BB, 1, 1), lambda g, n: (g, 0, 0)),
        out_shape=jax.ShapeDtypeStruct((B, 1, 1), jnp.float32),
        scratch_shapes=[
            pltpu.VMEM((_BB, T, 1), jnp.float32),
            pltpu.VMEM((_BB, T, 1), jnp.int32),
            pltpu.VMEM((_BB, 1, _CHUNK), jnp.float32),
            pltpu.VMEM((C, _BB * T), jnp.float32),
        ],
        compiler_params=pltpu.CompilerParams(
            dimension_semantics=("parallel", "arbitrary"),
        ),
    )(tcp, targets)
    return jnp.sum(per_sample) / B


# speedup vs baseline: 3.1872x; 1.0017x over previous
import jax
import jax.numpy as jnp
from jax.experimental import pallas as pl
from jax.experimental.pallas import tpu as pltpu

_LAMBDA_BOX = 5.0
_CHUNK = 4096
_BB = 8


def _dl_kernel(tcp_ref, tgt_ref, out_ref, rmax_s, ridx_s, sp_s, cap_s):
    C = tcp_ref.shape[0]
    T = tgt_ref.shape[1]
    TB = _BB * T
    nb = pl.program_id(1)
    num_nb = pl.num_programs(1)

    @pl.when(nb == 0)
    def _init():
        rmax_s[:, :, :] = jnp.full((_BB, T, 1), -jnp.inf, jnp.float32)
        ridx_s[:, :, :] = jnp.zeros((_BB, T, 1), jnp.int32)
        sp_s[:, :] = jnp.zeros((_BB, _CHUNK), jnp.float32)

    lane_tc = jax.lax.broadcasted_iota(jnp.int32, (T, _CHUNK), 1)

    cf_all = tcp_ref[4]
    sp_s[:, :] = (sp_s[:, :] + jnp.maximum(cf_all, 0.0)
                  + jnp.log1p(jnp.exp(-jnp.abs(cf_all))))
    ap_all = ((tcp_ref[2] - tcp_ref[0])
              * (tcp_ref[3] - tcp_ref[1]))

    onehots = []
    upds = []
    for bb in range(_BB):
        tgt = tgt_ref[bb]
        tx1 = tgt[:, 0:1]
        ty1 = tgt[:, 1:2]
        tx2 = tgt[:, 2:3]
        ty2 = tgt[:, 3:4]
        area_t = (tx2 - tx1) * (ty2 - ty1)

        px1 = tcp_ref[0:1, bb, :]
        py1 = tcp_ref[1:2, bb, :]
        px2 = tcp_ref[2:3, bb, :]
        py2 = tcp_ref[3:4, bb, :]
        w = jnp.maximum(jnp.minimum(px2, tx2) - jnp.maximum(px1, tx1), 0.0)
        h = jnp.maximum(jnp.minimum(py2, ty2) - jnp.maximum(py1, ty1), 0.0)
        inter = w * h
        r = inter / (ap_all[bb:bb + 1, :] + area_t)
        lmax = jnp.max(r, axis=1, keepdims=True)
        larg = jnp.argmax(r, axis=1, keepdims=True).astype(jnp.int32)
        upd = lmax > rmax_s[bb]
        rmax_s[bb] = jnp.where(upd, lmax, rmax_s[bb])
        ridx_s[bb] = jnp.where(upd, larg + nb * _CHUNK, ridx_s[bb])
        onehots.append(jnp.where(lane_tc == larg, 1.0, 0.0))
        upds.append(jnp.swapaxes(jnp.where(upd, 1.0, 0.0), 0, 1))

    blk2d = jnp.reshape(tcp_ref[:, :, :], (C * _BB, _CHUNK))
    oh = jnp.concatenate(onehots, axis=0)
    capmm = jax.lax.dot_general(
        blk2d, oh, (((1,), (1,)), ((), ())),
        preferred_element_type=jnp.float32)
    cap3 = jnp.reshape(capmm, (C, _BB, TB))
    bsel = (jax.lax.broadcasted_iota(jnp.int32, (_BB, TB), 1) // T
            == jax.lax.broadcasted_iota(jnp.int32, (_BB, TB), 0))
    cap_new = jnp.sum(jnp.where(bsel[None], cap3, 0.0), axis=1)
    upd_all = jnp.concatenate(upds, axis=1) > 0.0
    cap_s[:, :] = jnp.where(upd_all, cap_new, cap_s[:, :])

    @pl.when(nb == num_nb - 1)
    def _finish():
        for bb in range(_BB):
            tgt_t = jnp.swapaxes(tgt_ref[bb], 0, 1)
            g = cap_s[:, bb * T:(bb + 1) * T]

            d = jnp.abs(g[0:4, :] - tgt_t[0:4, :])
            box_loss = jnp.sum(jnp.where(d < 1.0, 0.5 * d * d, d - 0.5),
                               axis=(0, 1), keepdims=True)

            logits = g[5:, :]
            m = jnp.max(logits, axis=0, keepdims=True)
            lse = m + jnp.log(jnp.sum(jnp.exp(logits - m), axis=0,
                                      keepdims=True))
            tcls = tgt_t[4:5, :].astype(jnp.int32)
            cls_iota = jax.lax.broadcasted_iota(jnp.int32, (C - 5, T), 0)
            logit_t = jnp.sum(jnp.where(cls_iota == tcls, logits, 0.0),
                              axis=0, keepdims=True)
            cls_loss = jnp.sum(lse - logit_t, axis=(0, 1), keepdims=True)

            x = g[4:5, :]
            ri_col = ridx_s[bb]
            ri_row = jnp.swapaxes(ri_col, 0, 1)
            eq = ri_col == ri_row
            si = jax.lax.broadcasted_iota(jnp.int32, (T, T), 0)
            ti = jax.lax.broadcasted_iota(jnp.int32, (T, T), 1)
            dup = jnp.sum(jnp.where(eq & (si < ti), 1.0, 0.0), axis=0,
                          keepdims=True) > 0.0
            conf_sub = jnp.sum(jnp.where(dup, 0.0, x), axis=(0, 1),
                               keepdims=True)
            sp_total = jnp.sum(sp_s[bb:bb + 1, :], axis=(0, 1),
                               keepdims=True)

            out_ref[bb] = (_LAMBDA_BOX * box_loss + cls_loss
                           + sp_total - conf_sub)


def kernel(raw_preds, targets, epoch):
    del epoch
    B, P, C = raw_preds.shape
    T = targets.shape[1]
    nb = P // _CHUNK
    tcp = jnp.transpose(raw_preds, (2, 0, 1))
    per_sample = pl.pallas_call(
        _dl_kernel,
        grid=(B // _BB, nb),
        in_specs=[
            pl.BlockSpec((C, _BB, _CHUNK), lambda g, n: (0, g, n)),
            pl.BlockSpec((_BB, T, 5), lambda g, n: (g, 0, 0)),
        ],
        out_specs=pl.BlockSpec((_BB, 1, 1), lambda g, n: (g, 0, 0)),
        out_shape=jax.ShapeDtypeStruct((B, 1, 1), jnp.float32),
        scratch_shapes=[
            pltpu.VMEM((_BB, T, 1), jnp.float32),
            pltpu.VMEM((_BB, T, 1), jnp.int32),
            pltpu.VMEM((_BB, _CHUNK), jnp.float32),
            pltpu.VMEM((C, _BB * T), jnp.float32),
        ],
        compiler_params=pltpu.CompilerParams(
            dimension_semantics=("parallel", "arbitrary"),
        ),
    )(tcp, targets)
    return jnp.sum(per_sample) / B
